# Initial kernel scaffold; baseline (speedup 1.0000x reference)
import jax
import jax.numpy as jnp
from jax.experimental import pallas as pl


def kernel(x, edge_index, W1, b1, W2, b2, W3, b3, W4, b4):
    raise NotImplementedError("write your pallas kernel here")


# speedup vs baseline: 4.9657x; 4.9657x over previous
import functools

import jax
import jax.numpy as jnp
from jax import lax
from jax.experimental import pallas as pl
from jax.experimental.pallas import tpu as pltpu
from jax.experimental.pallas import tpu_sc as plsc

N = 10000
D = 128
E = 320000
NC = 2
NS = 16
NW = NC * NS
BATCH = 128
KC = 16
NB = 80
NBC = NB // KC
EPAD = NW * NB * BATCH
NPAD = N + 112
SLICE = NPAD // NS
DW = 16

_mesh = plsc.VectorSubcoreMesh(core_axis_name="c", subcore_axis_name="s")


def _sc_agg_body(hs, rows, cols, zsl, out, rows_v, cols_v, buf0, buf1, acc, g0, g1):
    cid = lax.axis_index("c")
    sid = lax.axis_index("s")
    wid = sid * NC + cid
    pltpu.sync_copy(zsl, acc.at[pl.ds(sid * SLICE, SLICE)])
    plsc.subcore_barrier()

    bufs = (buf0, buf1)
    sems = (g0, g1)

    def chunk_body(ci, _):
        pltpu.sync_copy(rows.at[wid, pl.ds(ci * KC, KC)], rows_v)
        pltpu.sync_copy(cols.at[wid, pl.ds(ci * KC, KC)], cols_v)
        pltpu.async_copy(hs.at[rows_v.at[0]], buf0, g0)
        for b in range(KC):
            cur, curg = bufs[b % 2], sems[b % 2]
            if b + 1 < KC:
                pltpu.async_copy(hs.at[rows_v.at[b + 1]],
                                 bufs[(b + 1) % 2], sems[(b + 1) % 2])
            pltpu.make_async_copy(hs.at[rows_v.at[b]], cur, curg).wait()
            pltpu.sync_copy(cur, acc.at[cols_v.at[b]], add=True)
        return 0

    lax.fori_loop(0, NBC, chunk_body, 0)
    plsc.subcore_barrier()
    pltpu.sync_copy(acc.at[pl.ds(sid * SLICE, SLICE)],
                    out.at[cid, pl.ds(sid * SLICE, SLICE)])


_sc_agg = functools.partial(
    pl.kernel,
    out_type=jax.ShapeDtypeStruct((NC, NPAD, D), jnp.float32),
    mesh=_mesh,
    scratch_types=[
        pltpu.VMEM((KC, BATCH), jnp.int32),
        pltpu.VMEM((KC, BATCH), jnp.int32),
        pltpu.VMEM((BATCH, D), jnp.float32),
        pltpu.VMEM((BATCH, D), jnp.float32),
        pltpu.VMEM_SHARED((NPAD, D), jnp.float32),
        pltpu.SemaphoreType.DMA,
        pltpu.SemaphoreType.DMA,
    ],
)(_sc_agg_body)


def _sc_deg_body(cols, ones, zsl, out, cols_v, ones_v, acc):
    cid = lax.axis_index("c")
    sid = lax.axis_index("s")
    wid = sid * NC + cid
    pltpu.sync_copy(zsl, acc.at[pl.ds(sid * SLICE, SLICE)])
    pltpu.sync_copy(cols.at[wid], cols_v)
    pltpu.sync_copy(ones, ones_v)
    plsc.subcore_barrier()

    def body(j, _):
        pltpu.sync_copy(ones_v, acc.at[cols_v.at[j]], add=True)
        return 0

    lax.fori_loop(0, NB, body, 0)
    plsc.subcore_barrier()
    pltpu.sync_copy(acc.at[pl.ds(sid * SLICE, SLICE)],
                    out.at[cid, pl.ds(sid * SLICE, SLICE)])


_sc_deg = functools.partial(
    pl.kernel,
    out_type=jax.ShapeDtypeStruct((NC, NPAD, D), jnp.float32),
    mesh=_mesh,
    scratch_types=[
        pltpu.VMEM((NB, BATCH), jnp.int32),
        pltpu.VMEM((BATCH, D), jnp.float32),
        pltpu.VMEM_SHARED((NPAD, D), jnp.float32),
    ],
)(_sc_deg_body)


BM = 1000


def _dinv(dg0, dg1):
    dsum = dg0[...] + dg1[...]
    return jnp.where(dsum > 0, lax.rsqrt(dsum), 0.0)


def _tc_first_body(x, wt, dg0, dg1, o):
    d = _dinv(dg0, dg1)
    o[...] = d * jnp.dot(x[...], wt[...], preferred_element_type=jnp.float32,
                         precision=lax.Precision.HIGHEST)


def _tc_mid_body(a0, a1, dg0, dg1, b, wt, o):
    d = _dinv(dg0, dg1)
    g = jnp.maximum(d * (a0[...] + a1[...]) + b[...], 0.0)
    o[...] = d * jnp.dot(g, wt[...], preferred_element_type=jnp.float32,
                         precision=lax.Precision.HIGHEST)


def _tc_last_body(a0, a1, dg0, dg1, b, o):
    d = _dinv(dg0, dg1)
    o[...] = d * (a0[...] + a1[...]) + b[...]


_row_spec = pl.BlockSpec((BM, D), lambda i: (i, 0))
_deg_spec = pl.BlockSpec((BM, 1), lambda i: (i, 0))
_w_spec = pl.BlockSpec((D, D), lambda i: (0, 0))
_b_spec = pl.BlockSpec((1, D), lambda i: (0, 0))
_out_sds = jax.ShapeDtypeStruct((N, D), jnp.float32)


def _tc_first(x, wt, dg0, dg1):
    return pl.pallas_call(
        _tc_first_body, grid=(N // BM,),
        in_specs=[_row_spec, _w_spec, _deg_spec, _deg_spec],
        out_specs=_row_spec, out_shape=_out_sds,
    )(x, wt, dg0, dg1)


def _tc_mid(a0, a1, dg0, dg1, b, wt):
    return pl.pallas_call(
        _tc_mid_body, grid=(N // BM,),
        in_specs=[_row_spec, _row_spec, _deg_spec, _deg_spec, _b_spec, _w_spec],
        out_specs=_row_spec, out_shape=_out_sds,
    )(a0, a1, dg0, dg1, b, wt)


def _tc_last(a0, a1, dg0, dg1, b):
    return pl.pallas_call(
        _tc_last_body, grid=(N // BM,),
        in_specs=[_row_spec, _row_spec, _deg_spec, _deg_spec, _b_spec],
        out_specs=_row_spec, out_shape=_out_sds,
    )(a0, a1, dg0, dg1, b)


def kernel(x, edge_index, W1, b1, W2, b2, W3, b3, W4, b4):
    row = edge_index[0].astype(jnp.int32)
    col = edge_index[1].astype(jnp.int32)
    rows3 = jnp.pad(row, (0, EPAD - E)).reshape(NW, NB, BATCH)
    cols3 = jnp.pad(col, (0, EPAD - E), constant_values=N).reshape(NW, NB, BATCH)
    zsl = jnp.zeros((SLICE, D), jnp.float32)
    ones = jnp.ones((BATCH, D), jnp.float32)

    degp = _sc_deg(cols3, ones, zsl)
    dg0 = degp[0, :N, :1]
    dg1 = degp[1, :N, :1]

    hs = _tc_first(x, W1.T, dg0, dg1)
    agg = _sc_agg(hs, rows3, cols3, zsl)
    hs = _tc_mid(agg[0, :N], agg[1, :N], dg0, dg1, b1.reshape(1, D), W2.T)
    agg = _sc_agg(hs, rows3, cols3, zsl)
    hs = _tc_mid(agg[0, :N], agg[1, :N], dg0, dg1, b2.reshape(1, D), W3.T)
    agg = _sc_agg(hs, rows3, cols3, zsl)
    hs = _tc_mid(agg[0, :N], agg[1, :N], dg0, dg1, b3.reshape(1, D), W4.T)
    agg = _sc_agg(hs, rows3, cols3, zsl)
    return _tc_last(agg[0, :N], agg[1, :N], dg0, dg1, b4.reshape(1, D))


# speedup vs baseline: 5.2694x; 1.0612x over previous
import functools

import jax
import jax.numpy as jnp
from jax import lax
from jax.experimental import pallas as pl
from jax.experimental.pallas import tpu as pltpu
from jax.experimental.pallas import tpu_sc as plsc

N = 10000
D = 128
E = 320000
NC = 2
NS = 16
NW = NC * NS
BATCH = 128
KC = 16
NB = 80
NBC = NB // KC
EPAD = NW * NB * BATCH
TB = NW * NB
CB0 = 128
CB1 = 32
NPAD = N + 112
SLICE = NPAD // NS
DW = 16

_mesh = plsc.VectorSubcoreMesh(core_axis_name="c", subcore_axis_name="s")


def _sc_agg_body(hs, rows, cols, zsl, out, rows_v, cols_v, buf0, buf1, acc, g0, g1):
    cid = lax.axis_index("c")
    sid = lax.axis_index("s")
    pltpu.sync_copy(zsl, acc.at[pl.ds(sid * SLICE, SLICE)])
    plsc.subcore_barrier()

    bufs = (buf0, buf1)
    sems = (g0, g1)
    start = lax.select(cid == 0, sid * CB0, NS * CB0 + sid * CB1)
    nchunks = lax.select(cid == 0, CB0 // KC, CB1 // KC)

    def chunk_body(ci, _):
        pltpu.sync_copy(rows.at[pl.ds(start + ci * KC, KC)], rows_v)
        pltpu.sync_copy(cols.at[pl.ds(start + ci * KC, KC)], cols_v)
        pltpu.async_copy(hs.at[rows_v.at[0]], buf0, g0)
        for b in range(KC):
            cur, curg = bufs[b % 2], sems[b % 2]
            if b + 1 < KC:
                pltpu.async_copy(hs.at[rows_v.at[b + 1]],
                                 bufs[(b + 1) % 2], sems[(b + 1) % 2])
            pltpu.make_async_copy(hs.at[rows_v.at[b]], cur, curg).wait()
            pltpu.sync_copy(cur, acc.at[cols_v.at[b]], add=True)
        return 0

    lax.fori_loop(0, nchunks, chunk_body, 0)
    plsc.subcore_barrier()
    pltpu.sync_copy(acc.at[pl.ds(sid * SLICE, SLICE)],
                    out.at[cid, pl.ds(sid * SLICE, SLICE)])


_sc_agg = functools.partial(
    pl.kernel,
    out_type=jax.ShapeDtypeStruct((NC, NPAD, D), jnp.float32),
    mesh=_mesh,
    scratch_types=[
        pltpu.VMEM((KC, BATCH), jnp.int32),
        pltpu.VMEM((KC, BATCH), jnp.int32),
        pltpu.VMEM((BATCH, D), jnp.float32),
        pltpu.VMEM((BATCH, D), jnp.float32),
        pltpu.VMEM_SHARED((NPAD, D), jnp.float32),
        pltpu.SemaphoreType.DMA,
        pltpu.SemaphoreType.DMA,
    ],
)(_sc_agg_body)


def _sc_deg_body(cols, ones, zsl, out, cols_v, ones_v, acc):
    cid = lax.axis_index("c")
    sid = lax.axis_index("s")
    wid = sid * NC + cid
    pltpu.sync_copy(zsl, acc.at[pl.ds(sid * SLICE, SLICE)])
    pltpu.sync_copy(cols.at[pl.ds(wid * NB, NB)], cols_v)
    pltpu.sync_copy(ones, ones_v)
    plsc.subcore_barrier()

    def body(j, _):
        pltpu.sync_copy(ones_v, acc.at[cols_v.at[j]], add=True)
        return 0

    lax.fori_loop(0, NB, body, 0)
    plsc.subcore_barrier()
    pltpu.sync_copy(acc.at[pl.ds(sid * SLICE, SLICE)],
                    out.at[cid, pl.ds(sid * SLICE, SLICE)])


_sc_deg = functools.partial(
    pl.kernel,
    out_type=jax.ShapeDtypeStruct((NC, NPAD, D), jnp.float32),
    mesh=_mesh,
    scratch_types=[
        pltpu.VMEM((NB, BATCH), jnp.int32),
        pltpu.VMEM((BATCH, D), jnp.float32),
        pltpu.VMEM_SHARED((NPAD, D), jnp.float32),
    ],
)(_sc_deg_body)


BM = 1000


def _dinv(dg0, dg1):
    dsum = dg0[...] + dg1[...]
    return jnp.where(dsum > 0, lax.rsqrt(dsum), 0.0)


def _tc_first_body(x, wt, dg0, dg1, o):
    d = _dinv(dg0, dg1)
    o[...] = d * jnp.dot(x[...], wt[...], preferred_element_type=jnp.float32,
                         precision=lax.Precision.HIGHEST)


def _tc_mid_body(a0, a1, dg0, dg1, b, wt, o):
    d = _dinv(dg0, dg1)
    g = jnp.maximum(d * (a0[...] + a1[...]) + b[...], 0.0)
    o[...] = d * jnp.dot(g, wt[...], preferred_element_type=jnp.float32,
                         precision=lax.Precision.HIGHEST)


def _tc_last_body(a0, a1, dg0, dg1, b, o):
    d = _dinv(dg0, dg1)
    o[...] = d * (a0[...] + a1[...]) + b[...]


_row_spec = pl.BlockSpec((BM, D), lambda i: (i, 0))
_deg_spec = pl.BlockSpec((BM, 1), lambda i: (i, 0))
_w_spec = pl.BlockSpec((D, D), lambda i: (0, 0))
_b_spec = pl.BlockSpec((1, D), lambda i: (0, 0))
_out_sds = jax.ShapeDtypeStruct((N, D), jnp.float32)


def _tc_first(x, wt, dg0, dg1):
    return pl.pallas_call(
        _tc_first_body, grid=(N // BM,),
        in_specs=[_row_spec, _w_spec, _deg_spec, _deg_spec],
        out_specs=_row_spec, out_shape=_out_sds,
    )(x, wt, dg0, dg1)


def _tc_mid(a0, a1, dg0, dg1, b, wt):
    return pl.pallas_call(
        _tc_mid_body, grid=(N // BM,),
        in_specs=[_row_spec, _row_spec, _deg_spec, _deg_spec, _b_spec, _w_spec],
        out_specs=_row_spec, out_shape=_out_sds,
    )(a0, a1, dg0, dg1, b, wt)


def _tc_last(a0, a1, dg0, dg1, b):
    return pl.pallas_call(
        _tc_last_body, grid=(N // BM,),
        in_specs=[_row_spec, _row_spec, _deg_spec, _deg_spec, _b_spec],
        out_specs=_row_spec, out_shape=_out_sds,
    )(a0, a1, dg0, dg1, b)


def kernel(x, edge_index, W1, b1, W2, b2, W3, b3, W4, b4):
    row = edge_index[0].astype(jnp.int32)
    col = edge_index[1].astype(jnp.int32)
    rows3 = jnp.pad(row, (0, EPAD - E)).reshape(TB, BATCH)
    cols3 = jnp.pad(col, (0, EPAD - E), constant_values=N).reshape(TB, BATCH)
    zsl = jnp.zeros((SLICE, D), jnp.float32)
    ones = jnp.ones((BATCH, D), jnp.float32)

    degp = _sc_deg(cols3, ones, zsl)
    dg0 = degp[0, :N, :1]
    dg1 = degp[1, :N, :1]

    hs = _tc_first(x, W1.T, dg0, dg1)
    agg = _sc_agg(hs, rows3, cols3, zsl)
    hs = _tc_mid(agg[0, :N], agg[1, :N], dg0, dg1, b1.reshape(1, D), W2.T)
    agg = _sc_agg(hs, rows3, cols3, zsl)
    hs = _tc_mid(agg[0, :N], agg[1, :N], dg0, dg1, b2.reshape(1, D), W3.T)
    agg = _sc_agg(hs, rows3, cols3, zsl)
    hs = _tc_mid(agg[0, :N], agg[1, :N], dg0, dg1, b3.reshape(1, D), W4.T)
    agg = _sc_agg(hs, rows3, cols3, zsl)
    return _tc_last(agg[0, :N], agg[1, :N], dg0, dg1, b4.reshape(1, D))
